# Initial kernel scaffold; baseline (speedup 1.0000x reference)
import jax
import jax.numpy as jnp
from jax.experimental import pallas as pl


def kernel(features, edge_index, W_self0, W_neigh0, b0, W_self1, W_neigh1, b1, W_self2, W_neigh2, b2):
    raise NotImplementedError("write your pallas kernel here")


# speedup vs baseline: 2.9835x; 2.9835x over previous
import functools

import jax
import jax.numpy as jnp
from jax import lax
from jax.experimental import pallas as pl
from jax.experimental.pallas import tpu as pltpu
from jax.experimental.pallas import tpu_sc as plsc

N = 10000
E = 320000
D = 128
DH = 64
H2 = 64

NC = 2
NS = 16
NW = NC * NS
NPAD = 10240
RPT = NPAD // NS
C = 128
EPAD = NW * 10240
K = EPAD // (NW * C)
UNROLL = 4
WCH = RPT // C


def _sc_agg_body(with_deg, x_lo, x_hi, src_h, dst_h, zrows_h, zrow1_h,
                 ones_h, *refs):
    if with_deg:
        aglo_o, aghi_o, deg_o = refs[0], refs[1], refs[2]
        (src_v, dst_v, r0, r1, r2, r3, zbuf, ones_v, zbuf1,
         acc, dega, s0, s1, s2, s3) = refs[3:]
    else:
        aglo_o, aghi_o = refs[0], refs[1]
        deg_o = None
        (src_v, dst_v, r0, r1, r2, r3, zbuf,
         acc, s0, s1, s2, s3) = refs[2:]
    c = lax.axis_index("c")
    s = lax.axis_index("s")
    wid = c * NS + s

    pltpu.sync_copy(src_h.at[pl.ds(wid * K, K)], src_v)
    pltpu.sync_copy(dst_h.at[pl.ds(wid * K, K)], dst_v)
    pltpu.sync_copy(zrows_h, zbuf)
    if with_deg:
        pltpu.sync_copy(zrow1_h, zbuf1)
        pltpu.sync_copy(ones_h, ones_v)

    bufs = (r0, r1, r2, r3)
    sems = (s0, s1, s2, s3)

    for half in range(2):
        x_h = (x_lo, x_hi)[half]
        out_h = (aglo_o, aghi_o)[half]
        first = half == 0

        for j in range(WCH):
            pltpu.sync_copy(zbuf, acc.at[pl.ds(s * RPT + j * C, C)])
            if with_deg and first:
                pltpu.sync_copy(zbuf1, dega.at[pl.ds(s * RPT + j * C, C)])
        plsc.subcore_barrier()

        def step(t, carry):
            k0 = t * UNROLL
            handles = []
            for u in range(UNROLL):
                handles.append(pltpu.async_copy(
                    x_h.at[src_v.at[k0 + u]], bufs[u], sems[u]))
            for u in range(UNROLL):
                handles[u].wait()
                pltpu.sync_copy(bufs[u], acc.at[dst_v.at[k0 + u]],
                                add=True)
                if with_deg and first:
                    pltpu.sync_copy(ones_v, dega.at[dst_v.at[k0 + u]],
                                    add=True)
            return carry

        lax.fori_loop(0, K // UNROLL, step, 0)
        plsc.subcore_barrier()

        for j in range(WCH):
            sl = pl.ds(s * RPT + j * C, C)
            pltpu.sync_copy(acc.at[sl], r0)
            pltpu.sync_copy(r0, out_h.at[c, sl])
            if with_deg and first:
                pltpu.sync_copy(dega.at[sl], ones_v)
                pltpu.sync_copy(ones_v, deg_o.at[c, sl])


@functools.cache
def _make_sc_agg(with_deg):
    out_type = [jax.ShapeDtypeStruct((NC, NPAD, DH), jnp.float32),
                jax.ShapeDtypeStruct((NC, NPAD, DH), jnp.float32)]
    if with_deg:
        out_type.append(jax.ShapeDtypeStruct((NC, NPAD), jnp.float32))
    mesh = plsc.VectorSubcoreMesh(
        core_axis_name="c", subcore_axis_name="s",
        num_cores=NC, num_subcores=NS)
    scratch = [
        pltpu.VMEM((K, C), jnp.int32),
        pltpu.VMEM((K, C), jnp.int32),
        pltpu.VMEM((C, DH), jnp.float32),
        pltpu.VMEM((C, DH), jnp.float32),
        pltpu.VMEM((C, DH), jnp.float32),
        pltpu.VMEM((C, DH), jnp.float32),
        pltpu.VMEM((C, DH), jnp.float32),
    ]
    if with_deg:
        scratch += [
            pltpu.VMEM((C,), jnp.float32),
            pltpu.VMEM((C,), jnp.float32),
        ]
    scratch += [pltpu.VMEM_SHARED((NPAD, DH), jnp.float32)]
    if with_deg:
        scratch += [pltpu.VMEM_SHARED((NPAD,), jnp.float32)]
    scratch += [pltpu.SemaphoreType.DMA] * 4
    return pl.kernel(
        functools.partial(_sc_agg_body, with_deg),
        out_type=out_type, mesh=mesh, scratch_types=scratch,
        compiler_params=pltpu.CompilerParams(use_tc_tiling_on_sc=False))


def _tc_layer0(x_ref, alo0_ref, alo1_ref, ahi0_ref, ahi1_ref,
               d0_ref, d1_ref, ws_ref, wn_ref, b_ref,
               olo_ref, ohi_ref):
    inv = 1.0 / jnp.maximum(d0_ref[...] + d1_ref[...], 1.0)
    hn = jnp.concatenate(
        [(alo0_ref[...] + alo1_ref[...]) * inv,
         (ahi0_ref[...] + ahi1_ref[...]) * inv], axis=1)
    acc = jnp.dot(x_ref[...], ws_ref[...],
                  preferred_element_type=jnp.float32,
                  precision=lax.Precision.HIGHEST)
    acc += jnp.dot(hn, wn_ref[...],
                   preferred_element_type=jnp.float32,
                   precision=lax.Precision.HIGHEST)
    h = jnp.maximum(acc + b_ref[...], 0.0)
    olo_ref[...] = h[:, :DH]
    ohi_ref[...] = h[:, DH:]


def _tc_layer12(hlo_ref, hhi_ref, alo0_ref, alo1_ref, ahi0_ref, ahi1_ref,
                d0_ref, d1_ref, ws1_ref, wn1_ref, b1_ref,
                ws2_ref, wn2_ref, b2_ref, noise_ref, o_ref):
    inv = 1.0 / jnp.maximum(d0_ref[...] + d1_ref[...], 1.0)
    hn = jnp.concatenate(
        [(alo0_ref[...] + alo1_ref[...]) * inv,
         (ahi0_ref[...] + ahi1_ref[...]) * inv], axis=1)
    h = jnp.concatenate([hlo_ref[...], hhi_ref[...]], axis=1)
    mm = functools.partial(jnp.dot, preferred_element_type=jnp.float32,
                           precision=lax.Precision.HIGHEST)
    mean = mm(h, ws1_ref[...]) + mm(hn, wn1_ref[...]) + b1_ref[...]
    log_std = mm(h, ws2_ref[...]) + mm(hn, wn2_ref[...]) + b2_ref[...]
    o_ref[...] = mean + noise_ref[...] * jnp.exp(log_std)


def kernel(features, edge_index, W_self0, W_neigh0, b0,
           W_self1, W_neigh1, b1, W_self2, W_neigh2, b2):
    src = edge_index[0]
    dst = edge_index[1]
    pad = EPAD - E
    src_p = jnp.concatenate([src, jnp.zeros((pad,), jnp.int32)])
    dst_p = jnp.concatenate([dst, jnp.full((pad,), NPAD - 1, jnp.int32)])
    src2d = src_p.reshape(NW * K, C)
    dst2d = dst_p.reshape(NW * K, C)

    zeros_rows = jnp.zeros((C, DH), jnp.float32)
    zeros_row1 = jnp.zeros((C,), jnp.float32)
    ones_row = jnp.ones((C,), jnp.float32)

    x_lo = features[:, :DH]
    x_hi = features[:, DH:]
    aglo, aghi, deg_parts = _make_sc_agg(True)(
        x_lo, x_hi, src2d, dst2d, zeros_rows, zeros_row1, ones_row)
    d0 = deg_parts[0, :N, None]
    d1 = deg_parts[1, :N, None]

    BN = 1000
    row_blk = lambda w: pl.BlockSpec((BN, w), lambda i: (i, 0))
    full_blk = lambda r, w: pl.BlockSpec((r, w), lambda i: (0, 0))
    h_lo, h_hi = pl.pallas_call(
        _tc_layer0,
        grid=(N // BN,),
        in_specs=[row_blk(D), row_blk(DH), row_blk(DH), row_blk(DH),
                  row_blk(DH), row_blk(1), row_blk(1),
                  full_blk(D, D), full_blk(D, D), full_blk(1, D)],
        out_specs=[row_blk(DH), row_blk(DH)],
        out_shape=[jax.ShapeDtypeStruct((N, DH), jnp.float32),
                   jax.ShapeDtypeStruct((N, DH), jnp.float32)],
    )(features, aglo[0, :N], aglo[1, :N], aghi[0, :N], aghi[1, :N],
      d0, d1, W_self0, W_neigh0, b0[None, :])

    ahlo, ahhi = _make_sc_agg(False)(
        h_lo, h_hi, src2d, dst2d, zeros_rows, zeros_row1, ones_row)

    noise = jax.random.normal(jax.random.key(1), (N, H2), dtype=jnp.float32)
    z = pl.pallas_call(
        _tc_layer12,
        grid=(N // BN,),
        in_specs=[row_blk(DH), row_blk(DH), row_blk(DH), row_blk(DH),
                  row_blk(DH), row_blk(DH), row_blk(1), row_blk(1),
                  full_blk(D, H2), full_blk(D, H2), full_blk(1, H2),
                  full_blk(D, H2), full_blk(D, H2), full_blk(1, H2),
                  row_blk(H2)],
        out_specs=pl.BlockSpec((BN, H2), lambda i: (i, 0)),
        out_shape=jax.ShapeDtypeStruct((N, H2), jnp.float32),
    )(h_lo, h_hi, ahlo[0, :N], ahlo[1, :N], ahhi[0, :N], ahhi[1, :N],
      d0, d1, W_self1, W_neigh1, b1[None, :],
      W_self2, W_neigh2, b2[None, :], noise)
    return z
